# baseline (speedup 1.0000x reference)
import jax
import jax.numpy as jnp
from jax.experimental import pallas as pl


_B, _H, _W, _C = 4, 112, 112, 384
_HB = 8


def _unpool_body(x_ref, o_ref):
    x = x_ref[0]
    xr = x.reshape(_HB, 2, 56, _C)
    o_ref[0] = jnp.zeros((_HB, 2, _W, 2 * _C), jnp.float32)
    o_ref[0, :, :, 0:56, 0:_C] = xr


def kernel(inputs):
    grid = (_B, _H // _HB)
    out6 = pl.pallas_call(
        _unpool_body,
        grid=grid,
        in_specs=[
            pl.BlockSpec((1, _HB, _W, _C), lambda b, i: (b, i, 0, 0)),
        ],
        out_specs=pl.BlockSpec(
            (1, _HB, 2, _W, 2 * _C), lambda b, i: (b, i, 0, 0, 0)
        ),
        out_shape=jax.ShapeDtypeStruct((_B, _H, 2, _W, 2 * _C), jnp.float32),
    )(inputs)
    return out6.reshape(_B, 2 * _H, 2 * _W, _C)


# speedup vs baseline: 3.0792x; 3.0792x over previous
import jax
import jax.numpy as jnp
from jax.experimental import pallas as pl


_B, _H, _W, _C = 4, 112, 112, 384
_HB = 8


def _unpool_body(x_ref, o_ref):
    x = x_ref[0]
    xr = x.reshape(_HB, 2, 56, 1, _C)
    inter = jnp.concatenate([xr, jnp.zeros_like(xr)], axis=3)
    inter = inter.reshape(_HB, 2, _W, _C)
    padw = jnp.concatenate(
        [inter, jnp.zeros((_HB, 2, _W, _C), jnp.float32)], axis=2
    )
    o_ref[0] = padw.reshape(2 * _HB, 2 * _W, _C)


def kernel(inputs):
    grid = (_B, _H // _HB)
    return pl.pallas_call(
        _unpool_body,
        grid=grid,
        in_specs=[
            pl.BlockSpec((1, _HB, _W, _C), lambda b, i: (b, i, 0, 0)),
        ],
        out_specs=pl.BlockSpec(
            (1, 2 * _HB, 2 * _W, _C), lambda b, i: (b, i, 0, 0)
        ),
        out_shape=jax.ShapeDtypeStruct((_B, 2 * _H, 2 * _W, _C), jnp.float32),
    )(inputs)


# speedup vs baseline: 3.4703x; 1.1270x over previous
import jax
import jax.numpy as jnp
from jax.experimental import pallas as pl


_B, _H, _W, _C = 4, 112, 112, 384
_HB = 16


def _unpool_body(x_ref, o_ref):
    x = x_ref[0]
    xr = x.reshape(_HB, 2, 56, 1, _C)
    inter = jnp.concatenate([xr, jnp.zeros_like(xr)], axis=3)
    inter = inter.reshape(_HB, 2, _W, _C)
    padw = jnp.concatenate(
        [inter, jnp.zeros((_HB, 2, _W, _C), jnp.float32)], axis=2
    )
    o_ref[0] = padw.reshape(2 * _HB, 2 * _W, _C)


def kernel(inputs):
    grid = (_B, _H // _HB)
    return pl.pallas_call(
        _unpool_body,
        grid=grid,
        in_specs=[
            pl.BlockSpec((1, _HB, _W, _C), lambda b, i: (b, i, 0, 0)),
        ],
        out_specs=pl.BlockSpec(
            (1, 2 * _HB, 2 * _W, _C), lambda b, i: (b, i, 0, 0)
        ),
        out_shape=jax.ShapeDtypeStruct((_B, 2 * _H, 2 * _W, _C), jnp.float32),
    )(inputs)


# speedup vs baseline: 3.5075x; 1.0107x over previous
import jax
import jax.numpy as jnp
from jax.experimental import pallas as pl


_B, _H, _W, _C = 4, 112, 112, 384
_HB = 28


def _unpool_body(x_ref, o_ref):
    x = x_ref[0]
    xr = x.reshape(_HB, 2, 56, 1, _C)
    inter = jnp.concatenate([xr, jnp.zeros_like(xr)], axis=3)
    inter = inter.reshape(_HB, 2, _W, _C)
    padw = jnp.concatenate(
        [inter, jnp.zeros((_HB, 2, _W, _C), jnp.float32)], axis=2
    )
    o_ref[0] = padw.reshape(2 * _HB, 2 * _W, _C)


def kernel(inputs):
    grid = (_B, _H // _HB)
    return pl.pallas_call(
        _unpool_body,
        grid=grid,
        in_specs=[
            pl.BlockSpec((1, _HB, _W, _C), lambda b, i: (b, i, 0, 0)),
        ],
        out_specs=pl.BlockSpec(
            (1, 2 * _HB, 2 * _W, _C), lambda b, i: (b, i, 0, 0)
        ),
        out_shape=jax.ShapeDtypeStruct((_B, 2 * _H, 2 * _W, _C), jnp.float32),
    )(inputs)
